# Initial kernel scaffold; baseline (speedup 1.0000x reference)
import jax
import jax.numpy as jnp
from jax.experimental import pallas as pl


def kernel(x, edge_index, W1, b1, W2, b2):
    raise NotImplementedError("write your pallas kernel here")


# speedup vs baseline: 11.2809x; 11.2809x over previous
import functools

import jax
import jax.numpy as jnp
from jax import lax
from jax.experimental import pallas as pl
from jax.experimental.pallas import tpu as pltpu
from jax.experimental.pallas import tpu_sc as plsc

NC = 2
NS = 16
NW = NC * NS
LANES = 16

_mesh = functools.partial(
    plsc.VectorSubcoreMesh, core_axis_name="c", subcore_axis_name="s")


def _make_deg(n, e):
    epw = e // NW

    @functools.partial(
        pl.kernel,
        mesh=_mesh(),
        out_type=jax.ShapeDtypeStruct((NW, n), jnp.float32),
        scratch_types=[
            pltpu.VMEM((n,), jnp.float32),
            pltpu.VMEM((epw,), jnp.int32),
        ],
        compiler_params=pltpu.CompilerParams(needs_layout_passes=False),
    )
    def deg_kernel(dst_hbm, zeros_hbm, out_hbm, hist_v, idx_v):
        c = lax.axis_index("c")
        s = lax.axis_index("s")
        wid = s * NC + c
        pltpu.sync_copy(zeros_hbm, hist_v)
        pltpu.sync_copy(dst_hbm.at[pl.ds(wid * epw, epw)], idx_v)
        ones = jnp.ones((LANES,), jnp.float32)

        def body(j, carry):
            iv = idx_v[pl.ds(j * LANES, LANES)]
            plsc.addupdate_scatter(hist_v, [iv], ones)
            return carry

        lax.fori_loop(0, epw // LANES, body, 0)
        pltpu.sync_copy(hist_v, out_hbm.at[wid])

    return deg_kernel


def _make_agg(n, e, d):
    epw = e // NW
    k = 80
    nch = epw // k

    @functools.partial(
        pl.kernel,
        mesh=_mesh(),
        out_type=jax.ShapeDtypeStruct((NC, n, d), jnp.float32),
        scratch_types=[
            pltpu.VMEM((k,), jnp.int32),
            pltpu.VMEM((k,), jnp.int32),
            pltpu.VMEM((k, d), jnp.float32),
            pltpu.VMEM_SHARED((n, d), jnp.float32),
            pltpu.SemaphoreType.DMA,
        ],
        compiler_params=pltpu.CompilerParams(use_tc_tiling_on_sc=False),
    )
    def agg_kernel(g_hbm, src_hbm, dst_hbm, zeros_hbm, out_hbm,
                   srci, dsti, rows, acc, sem):
        c = lax.axis_index("c")
        s = lax.axis_index("s")
        wid = s * NC + c

        @pl.when(s == 0)
        def _():
            pltpu.sync_copy(zeros_hbm, acc)

        plsc.subcore_barrier()

        base0 = wid * epw

        def body(i, carry):
            b = base0 + i * k
            pltpu.sync_copy(src_hbm.at[pl.ds(b, k)], srci)
            pltpu.sync_copy(dst_hbm.at[pl.ds(b, k)], dsti)
            pltpu.async_copy(g_hbm.at[srci], rows, sem).wait()
            pltpu.sync_copy(rows, acc.at[dsti], add=True)
            return carry

        lax.fori_loop(0, nch, body, 0)
        plsc.subcore_barrier()

        @pl.when(s == 0)
        def _():
            pltpu.sync_copy(acc, out_hbm.at[c])

    return agg_kernel


def _prep_body(degp_ref, dbc_ref):
    deg = 1.0 + jnp.sum(degp_ref[...], axis=1, keepdims=True)
    dbc_ref[...] = jnp.broadcast_to(lax.rsqrt(deg), dbc_ref.shape)


def _mm1_body(x_ref, w_ref, d_ref, ga_ref, gb_ref):
    h = jnp.dot(x_ref[...], w_ref[...], preferred_element_type=jnp.float32)
    g = h * d_ref[...][:, :1]
    ga_ref[...] = g[:, :128]
    gb_ref[...] = g[:, 128:]


def _mid_body(pa_ref, pb_ref, ga_ref, gb_ref, d_ref, b1_ref, w2_ref, g2_ref):
    dcol = d_ref[...][:, :1]
    b1 = b1_ref[...]
    h1a = jnp.maximum(dcol * (pa_ref[0] + pa_ref[1] + ga_ref[...])
                      + b1[:, :128], 0.0)
    h1b = jnp.maximum(dcol * (pb_ref[0] + pb_ref[1] + gb_ref[...])
                      + b1[:, 128:], 0.0)
    h2 = (jnp.dot(h1a, w2_ref[0], preferred_element_type=jnp.float32)
          + jnp.dot(h1b, w2_ref[1], preferred_element_type=jnp.float32))
    g2_ref[...] = h2 * dcol


def _fin_body(p2_ref, g2_ref, d_ref, b2_ref, out_ref):
    dcol = d_ref[...][:, :1]
    out_ref[...] = (dcol * (p2_ref[0] + p2_ref[1] + g2_ref[...])
                    + b2_ref[...])


def kernel(x, edge_index, W1, b1, W2, b2):
    n, df = x.shape
    e = edge_index.shape[1]
    h = W1.shape[1]
    do = W2.shape[1]
    ha = h // 2

    src = edge_index[0]
    dst = edge_index[1]

    deg_k = _make_deg(n, e)
    degp = deg_k(dst, jnp.zeros((n,), jnp.float32))

    bm = 1000
    grid = (n // bm,)

    dbc = pl.pallas_call(
        _prep_body,
        grid=grid,
        in_specs=[pl.BlockSpec((bm, NW), lambda i: (i, 0))],
        out_specs=pl.BlockSpec((bm, 8), lambda i: (i, 0)),
        out_shape=jax.ShapeDtypeStruct((n, 8), jnp.float32),
    )(degp.T)

    ga, gb = pl.pallas_call(
        _mm1_body,
        grid=grid,
        in_specs=[
            pl.BlockSpec((bm, df), lambda i: (i, 0)),
            pl.BlockSpec((df, h), lambda i: (0, 0)),
            pl.BlockSpec((bm, 8), lambda i: (i, 0)),
        ],
        out_specs=[
            pl.BlockSpec((bm, ha), lambda i: (i, 0)),
            pl.BlockSpec((bm, ha), lambda i: (i, 0)),
        ],
        out_shape=[
            jax.ShapeDtypeStruct((n, ha), jnp.float32),
            jax.ShapeDtypeStruct((n, ha), jnp.float32),
        ],
    )(x, W1, dbc)

    agg128 = _make_agg(n, e, ha)
    z128 = jnp.zeros((n, ha), jnp.float32)
    pa = agg128(ga, src, dst, z128)
    pb = agg128(gb, src, dst, z128)

    g2 = pl.pallas_call(
        _mid_body,
        grid=grid,
        in_specs=[
            pl.BlockSpec((NC, bm, ha), lambda i: (0, i, 0)),
            pl.BlockSpec((NC, bm, ha), lambda i: (0, i, 0)),
            pl.BlockSpec((bm, ha), lambda i: (i, 0)),
            pl.BlockSpec((bm, ha), lambda i: (i, 0)),
            pl.BlockSpec((bm, 8), lambda i: (i, 0)),
            pl.BlockSpec((1, h), lambda i: (0, 0)),
            pl.BlockSpec((2, ha, do), lambda i: (0, 0, 0)),
        ],
        out_specs=pl.BlockSpec((bm, do), lambda i: (i, 0)),
        out_shape=jax.ShapeDtypeStruct((n, do), jnp.float32),
    )(pa, pb, ga, gb, dbc, b1.reshape(1, h), W2.reshape(2, ha, do))

    agg64 = _make_agg(n, e, do)
    p2 = agg64(g2, src, dst, jnp.zeros((n, do), jnp.float32))

    out = pl.pallas_call(
        _fin_body,
        grid=grid,
        in_specs=[
            pl.BlockSpec((NC, bm, do), lambda i: (0, i, 0)),
            pl.BlockSpec((bm, do), lambda i: (i, 0)),
            pl.BlockSpec((bm, 8), lambda i: (i, 0)),
            pl.BlockSpec((1, do), lambda i: (0, 0)),
        ],
        out_specs=pl.BlockSpec((bm, do), lambda i: (i, 0)),
        out_shape=jax.ShapeDtypeStruct((n, do), jnp.float32),
    )(p2, g2, dbc, b2.reshape(1, do))

    return out


# speedup vs baseline: 20.3080x; 1.8002x over previous
import functools

import jax
import jax.numpy as jnp
from jax import lax
from jax.experimental import pallas as pl
from jax.experimental.pallas import tpu as pltpu
from jax.experimental.pallas import tpu_sc as plsc

NC = 2
NS = 16
NW = NC * NS
LANES = 16

_mesh = functools.partial(
    plsc.VectorSubcoreMesh, core_axis_name="c", subcore_axis_name="s")


def _make_deg(n, e):
    epw = e // NW

    @functools.partial(
        pl.kernel,
        mesh=_mesh(),
        out_type=jax.ShapeDtypeStruct((NW, n), jnp.float32),
        scratch_types=[
            pltpu.VMEM((n,), jnp.float32),
            pltpu.VMEM((epw,), jnp.int32),
        ],
        compiler_params=pltpu.CompilerParams(needs_layout_passes=False),
    )
    def deg_kernel(dst_hbm, zeros_hbm, out_hbm, hist_v, idx_v):
        c = lax.axis_index("c")
        s = lax.axis_index("s")
        wid = s * NC + c
        pltpu.sync_copy(zeros_hbm, hist_v)
        pltpu.sync_copy(dst_hbm.at[pl.ds(wid * epw, epw)], idx_v)
        ones = jnp.ones((LANES,), jnp.float32)

        def body(j, carry):
            iv = idx_v[pl.ds(j * LANES, LANES)]
            plsc.addupdate_scatter(hist_v, [iv], ones)
            return carry

        lax.fori_loop(0, epw // LANES, body, 0)
        pltpu.sync_copy(hist_v, out_hbm.at[wid])

    return deg_kernel


def _make_agg(n, e, d):
    epw = e // NW
    k = 80
    nch = epw // k
    rpt = n // NS

    @functools.partial(
        pl.kernel,
        mesh=_mesh(),
        out_type=jax.ShapeDtypeStruct((NC, n, d), jnp.float32),
        scratch_types=[
            pltpu.VMEM((nch, k), jnp.int32),
            pltpu.VMEM((nch, k), jnp.int32),
            pltpu.VMEM((k, d), jnp.float32),
            pltpu.VMEM((k, d), jnp.float32),
            pltpu.VMEM_SHARED((n, d), jnp.float32),
            pltpu.SemaphoreType.DMA,
            pltpu.SemaphoreType.DMA,
            pltpu.SemaphoreType.DMA,
            pltpu.SemaphoreType.DMA,
        ],
        compiler_params=pltpu.CompilerParams(use_tc_tiling_on_sc=False),
    )
    def agg_kernel(g_hbm, src3_hbm, dst3_hbm, zeros_hbm, out_hbm,
                   src_st, dst_st, rows0, rows1, acc, sg0, sg1, ss0, ss1):
        c = lax.axis_index("c")
        s = lax.axis_index("s")
        wid = s * NC + c

        pltpu.sync_copy(zeros_hbm.at[pl.ds(s * rpt, rpt)],
                        acc.at[pl.ds(s * rpt, rpt)])
        pltpu.sync_copy(src3_hbm.at[wid], src_st)
        pltpu.sync_copy(dst3_hbm.at[wid], dst_st)
        plsc.subcore_barrier()

        def gather_start(i, rows, sem):
            pltpu.async_copy(g_hbm.at[src_st.at[i]], rows, sem)

        def gather_wait(i, rows, sem):
            pltpu.make_async_copy(g_hbm.at[src_st.at[i]], rows, sem).wait()

        def scatter_start(i, rows, sem):
            pltpu.async_copy(rows, acc.at[dst_st.at[i]], sem, add=True)

        def scatter_wait(i, rows, sem):
            pltpu.make_async_copy(rows, acc.at[dst_st.at[i]], sem).wait()

        gather_start(0, rows0, sg0)

        def body(j, carry):
            i0 = 2 * j
            i1 = i0 + 1
            gather_wait(i0, rows0, sg0)

            @pl.when(j > 0)
            def _():
                scatter_wait(i1, rows1, ss1)

            gather_start(i1, rows1, sg1)
            scatter_start(i0, rows0, ss0)
            gather_wait(i1, rows1, sg1)
            scatter_wait(i0, rows0, ss0)

            @pl.when(i0 + 2 < nch)
            def _():
                gather_start(i0 + 2, rows0, sg0)

            scatter_start(i1, rows1, ss1)
            return carry

        lax.fori_loop(0, nch // 2, body, 0)
        last = nch - 1
        gather_wait(last, rows0, sg0)
        scatter_wait(last - 1, rows1, ss1)
        scatter_start(last, rows0, ss0)
        scatter_wait(last, rows0, ss0)
        plsc.subcore_barrier()

        pltpu.sync_copy(acc.at[pl.ds(s * rpt, rpt)],
                        out_hbm.at[c, pl.ds(s * rpt, rpt)])

    return agg_kernel


def _prep_body(degp_ref, dbc_ref):
    deg = 1.0 + jnp.sum(degp_ref[...], axis=1, keepdims=True)
    dbc_ref[...] = jnp.broadcast_to(lax.rsqrt(deg), dbc_ref.shape)


def _mm1_body(x_ref, w_ref, d_ref, ga_ref, gb_ref):
    h = jnp.dot(x_ref[...], w_ref[...], preferred_element_type=jnp.float32)
    g = h * d_ref[...][:, :1]
    ga_ref[...] = g[:, :128]
    gb_ref[...] = g[:, 128:]


def _mid_body(pa_ref, pb_ref, ga_ref, gb_ref, d_ref, b1_ref, w2_ref, g2_ref):
    dcol = d_ref[...][:, :1]
    b1 = b1_ref[...]
    h1a = jnp.maximum(dcol * (pa_ref[0] + pa_ref[1] + ga_ref[...])
                      + b1[:, :128], 0.0)
    h1b = jnp.maximum(dcol * (pb_ref[0] + pb_ref[1] + gb_ref[...])
                      + b1[:, 128:], 0.0)
    h2 = (jnp.dot(h1a, w2_ref[0], preferred_element_type=jnp.float32)
          + jnp.dot(h1b, w2_ref[1], preferred_element_type=jnp.float32))
    g2_ref[...] = h2 * dcol


def _fin_body(p2_ref, g2_ref, d_ref, b2_ref, out_ref):
    dcol = d_ref[...][:, :1]
    out_ref[...] = (dcol * (p2_ref[0] + p2_ref[1] + g2_ref[...])
                    + b2_ref[...])


def kernel(x, edge_index, W1, b1, W2, b2):
    n, df = x.shape
    e = edge_index.shape[1]
    h = W1.shape[1]
    do = W2.shape[1]
    ha = h // 2

    src = edge_index[0]
    dst = edge_index[1]
    k = 80
    src3 = src.reshape(NW, (e // NW) // k, k)
    dst3 = dst.reshape(NW, (e // NW) // k, k)

    deg_k = _make_deg(n, e)
    degp = deg_k(dst, jnp.zeros((n,), jnp.float32))

    bm = 1000
    grid = (n // bm,)

    dbc = pl.pallas_call(
        _prep_body,
        grid=grid,
        in_specs=[pl.BlockSpec((bm, NW), lambda i: (i, 0))],
        out_specs=pl.BlockSpec((bm, 8), lambda i: (i, 0)),
        out_shape=jax.ShapeDtypeStruct((n, 8), jnp.float32),
    )(degp.T)

    ga, gb = pl.pallas_call(
        _mm1_body,
        grid=grid,
        in_specs=[
            pl.BlockSpec((bm, df), lambda i: (i, 0)),
            pl.BlockSpec((df, h), lambda i: (0, 0)),
            pl.BlockSpec((bm, 8), lambda i: (i, 0)),
        ],
        out_specs=[
            pl.BlockSpec((bm, ha), lambda i: (i, 0)),
            pl.BlockSpec((bm, ha), lambda i: (i, 0)),
        ],
        out_shape=[
            jax.ShapeDtypeStruct((n, ha), jnp.float32),
            jax.ShapeDtypeStruct((n, ha), jnp.float32),
        ],
    )(x, W1, dbc)

    agg128 = _make_agg(n, e, ha)
    z128 = jnp.zeros((n, ha), jnp.float32)
    pa = agg128(ga, src3, dst3, z128)
    pb = agg128(gb, src3, dst3, z128)

    g2 = pl.pallas_call(
        _mid_body,
        grid=grid,
        in_specs=[
            pl.BlockSpec((NC, bm, ha), lambda i: (0, i, 0)),
            pl.BlockSpec((NC, bm, ha), lambda i: (0, i, 0)),
            pl.BlockSpec((bm, ha), lambda i: (i, 0)),
            pl.BlockSpec((bm, ha), lambda i: (i, 0)),
            pl.BlockSpec((bm, 8), lambda i: (i, 0)),
            pl.BlockSpec((1, h), lambda i: (0, 0)),
            pl.BlockSpec((2, ha, do), lambda i: (0, 0, 0)),
        ],
        out_specs=pl.BlockSpec((bm, do), lambda i: (i, 0)),
        out_shape=jax.ShapeDtypeStruct((n, do), jnp.float32),
    )(pa, pb, ga, gb, dbc, b1.reshape(1, h), W2.reshape(2, ha, do))

    agg64 = _make_agg(n, e, do)
    p2 = agg64(g2, src3, dst3, jnp.zeros((n, do), jnp.float32))

    out = pl.pallas_call(
        _fin_body,
        grid=grid,
        in_specs=[
            pl.BlockSpec((NC, bm, do), lambda i: (0, i, 0)),
            pl.BlockSpec((bm, do), lambda i: (i, 0)),
            pl.BlockSpec((bm, 8), lambda i: (i, 0)),
            pl.BlockSpec((1, do), lambda i: (0, 0)),
        ],
        out_specs=pl.BlockSpec((bm, do), lambda i: (i, 0)),
        out_shape=jax.ShapeDtypeStruct((n, do), jnp.float32),
    )(p2, g2, dbc, b2.reshape(1, do))

    return out


# speedup vs baseline: 24.2644x; 1.1948x over previous
import functools

import jax
import jax.numpy as jnp
from jax import lax
from jax.experimental import pallas as pl
from jax.experimental.pallas import tpu as pltpu
from jax.experimental.pallas import tpu_sc as plsc

NC = 2
NS = 16
NW = NC * NS
LANES = 16
K = 125


def _make_deg(n, e):
    epw = e // NW

    @functools.partial(
        pl.kernel,
        mesh=plsc.VectorSubcoreMesh(core_axis_name="c", subcore_axis_name="s"),
        out_type=jax.ShapeDtypeStruct((NW, n), jnp.float32),
        scratch_types=[
            pltpu.VMEM((n,), jnp.float32),
            pltpu.VMEM((epw,), jnp.int32),
        ],
        compiler_params=pltpu.CompilerParams(needs_layout_passes=False),
    )
    def deg_kernel(dst_hbm, zeros_hbm, out_hbm, hist_v, idx_v):
        c = lax.axis_index("c")
        s = lax.axis_index("s")
        wid = s * NC + c
        pltpu.sync_copy(zeros_hbm, hist_v)
        pltpu.sync_copy(dst_hbm.at[pl.ds(wid * epw, epw)], idx_v)
        ones = jnp.ones((LANES,), jnp.float32)

        def body(j, carry):
            iv = idx_v[pl.ds(j * LANES, LANES)]
            plsc.addupdate_scatter(hist_v, [iv], ones)
            return carry

        lax.fori_loop(0, epw // LANES, body, 0)
        pltpu.sync_copy(hist_v, out_hbm.at[wid])

    return deg_kernel


def _make_agg(n, e, d, ng):
    epw = e // NW
    nch = epw // K
    nh = 2
    nch_h = nch // nh
    rpt = n // NS

    def body_fn(g_hbms, ei_hbm, zeros_hbm, out_hbm,
                ist, rows0, rows1, acc, sg0, sg1, ss0, ss1):
        c = lax.axis_index("c")
        s = lax.axis_index("s")
        wid = s * NC + c
        rs = s * rpt

        for gi, g_hbm in enumerate(g_hbms):
            pltpu.sync_copy(zeros_hbm.at[pl.ds(rs, rpt)],
                            acc.at[pl.ds(rs, rpt)])
            plsc.subcore_barrier()

            def gather_start(i, rows, sem):
                pltpu.async_copy(g_hbm.at[ist.at[i, 0]], rows, sem)

            def gather_wait(i, rows, sem):
                pltpu.make_async_copy(
                    g_hbm.at[ist.at[i, 0]], rows, sem).wait()

            def scatter_start(i, rows, sem):
                pltpu.async_copy(rows, acc.at[ist.at[i, 1]], sem, add=True)

            def scatter_wait(i, rows, sem):
                pltpu.make_async_copy(
                    rows, acc.at[ist.at[i, 1]], sem).wait()

            for hh in range(nh):
                pltpu.sync_copy(
                    ei_hbm.at[wid, pl.ds(hh * nch_h, nch_h)], ist)
                gather_start(0, rows0, sg0)

                def body(j, carry):
                    i0 = 2 * j
                    i1 = i0 + 1
                    gather_wait(i0, rows0, sg0)

                    @pl.when(j > 0)
                    def _():
                        scatter_wait(i1, rows1, ss1)

                    gather_start(i1, rows1, sg1)
                    scatter_start(i0, rows0, ss0)
                    gather_wait(i1, rows1, sg1)
                    scatter_wait(i0, rows0, ss0)

                    @pl.when(i0 + 2 < nch_h)
                    def _():
                        gather_start(i0 + 2, rows0, sg0)

                    scatter_start(i1, rows1, ss1)
                    return carry

                lax.fori_loop(0, nch_h // 2, body, 0)
                scatter_wait(nch_h - 1, rows1, ss1)

            plsc.subcore_barrier()
            pltpu.sync_copy(acc.at[pl.ds(rs, rpt)],
                            out_hbm.at[gi, c, pl.ds(rs, rpt)])

    if ng == 1:
        def raw(g0, ei, z, out, ist, r0, r1, acc, sg0, sg1, ss0, ss1):
            body_fn([g0], ei, z, out, ist, r0, r1, acc, sg0, sg1, ss0, ss1)
    else:
        def raw(g0, g1, ei, z, out, ist, r0, r1, acc, sg0, sg1, ss0, ss1):
            body_fn([g0, g1], ei, z, out, ist, r0, r1, acc,
                    sg0, sg1, ss0, ss1)

    return pl.kernel(
        raw,
        mesh=plsc.VectorSubcoreMesh(core_axis_name="c", subcore_axis_name="s"),
        out_type=jax.ShapeDtypeStruct((ng, NC, n, d), jnp.float32),
        scratch_types=[
            pltpu.VMEM((nch_h, 2, K), jnp.int32),
            pltpu.VMEM((K, d), jnp.float32),
            pltpu.VMEM((K, d), jnp.float32),
            pltpu.VMEM_SHARED((n, d), jnp.float32),
            pltpu.SemaphoreType.DMA,
            pltpu.SemaphoreType.DMA,
            pltpu.SemaphoreType.DMA,
            pltpu.SemaphoreType.DMA,
        ],
        compiler_params=pltpu.CompilerParams(use_tc_tiling_on_sc=False),
    )


def _mm1_body(x_ref, w_ref, degp_ref, ga_ref, gb_ref, dbc_ref):
    deg = 1.0 + jnp.sum(degp_ref[...], axis=1, keepdims=True)
    d = lax.rsqrt(deg)
    h = jnp.dot(x_ref[...], w_ref[...], preferred_element_type=jnp.float32)
    g = h * d
    ga_ref[...] = g[:, :128]
    gb_ref[...] = g[:, 128:]
    dbc_ref[...] = jnp.broadcast_to(d, dbc_ref.shape)


def _mid_body(p1_ref, ga_ref, gb_ref, d_ref, b1_ref, w2_ref, g2_ref):
    dcol = d_ref[...][:, :1]
    b1 = b1_ref[...]
    h1a = jnp.maximum(dcol * (p1_ref[0, 0] + p1_ref[0, 1] + ga_ref[...])
                      + b1[:, :128], 0.0)
    h1b = jnp.maximum(dcol * (p1_ref[1, 0] + p1_ref[1, 1] + gb_ref[...])
                      + b1[:, 128:], 0.0)
    h2 = (jnp.dot(h1a, w2_ref[0], preferred_element_type=jnp.float32)
          + jnp.dot(h1b, w2_ref[1], preferred_element_type=jnp.float32))
    g2_ref[...] = h2 * dcol


def _fin_body(p2_ref, g2_ref, d_ref, b2_ref, out_ref):
    dcol = d_ref[...][:, :1]
    out_ref[...] = (dcol * (p2_ref[0, 0] + p2_ref[0, 1] + g2_ref[...])
                    + b2_ref[...])


def kernel(x, edge_index, W1, b1, W2, b2):
    n, df = x.shape
    e = edge_index.shape[1]
    h = W1.shape[1]
    do = W2.shape[1]
    ha = h // 2
    nch = (e // NW) // K

    src3 = edge_index[0].reshape(NW, nch, K)
    dst3 = edge_index[1].reshape(NW, nch, K)
    ei4 = jnp.stack([src3, dst3], axis=2)

    deg_k = _make_deg(n, e)
    degp = deg_k(edge_index[1], jnp.zeros((n,), jnp.float32))

    bm = 1000
    grid = (n // bm,)

    ga, gb, dbc = pl.pallas_call(
        _mm1_body,
        grid=grid,
        in_specs=[
            pl.BlockSpec((bm, df), lambda i: (i, 0)),
            pl.BlockSpec((df, h), lambda i: (0, 0)),
            pl.BlockSpec((bm, NW), lambda i: (i, 0)),
        ],
        out_specs=[
            pl.BlockSpec((bm, ha), lambda i: (i, 0)),
            pl.BlockSpec((bm, ha), lambda i: (i, 0)),
            pl.BlockSpec((bm, 8), lambda i: (i, 0)),
        ],
        out_shape=[
            jax.ShapeDtypeStruct((n, ha), jnp.float32),
            jax.ShapeDtypeStruct((n, ha), jnp.float32),
            jax.ShapeDtypeStruct((n, 8), jnp.float32),
        ],
    )(x, W1, degp.T)

    agg128 = _make_agg(n, e, ha, 2)
    p1 = agg128(ga, gb, ei4, jnp.zeros((n, ha), jnp.float32))

    g2 = pl.pallas_call(
        _mid_body,
        grid=grid,
        in_specs=[
            pl.BlockSpec((2, NC, bm, ha), lambda i: (0, 0, i, 0)),
            pl.BlockSpec((bm, ha), lambda i: (i, 0)),
            pl.BlockSpec((bm, ha), lambda i: (i, 0)),
            pl.BlockSpec((bm, 8), lambda i: (i, 0)),
            pl.BlockSpec((1, h), lambda i: (0, 0)),
            pl.BlockSpec((2, ha, do), lambda i: (0, 0, 0)),
        ],
        out_specs=pl.BlockSpec((bm, do), lambda i: (i, 0)),
        out_shape=jax.ShapeDtypeStruct((n, do), jnp.float32),
    )(p1, ga, gb, dbc, b1.reshape(1, h), W2.reshape(2, ha, do))

    agg64 = _make_agg(n, e, do, 1)
    p2 = agg64(g2, ei4, jnp.zeros((n, do), jnp.float32))

    out = pl.pallas_call(
        _fin_body,
        grid=grid,
        in_specs=[
            pl.BlockSpec((1, NC, bm, do), lambda i: (0, 0, i, 0)),
            pl.BlockSpec((bm, do), lambda i: (i, 0)),
            pl.BlockSpec((bm, 8), lambda i: (i, 0)),
            pl.BlockSpec((1, do), lambda i: (0, 0)),
        ],
        out_specs=pl.BlockSpec((bm, do), lambda i: (i, 0)),
        out_shape=jax.ShapeDtypeStruct((n, do), jnp.float32),
    )(p2, g2, dbc, b2.reshape(1, do))

    return out
